# baseline (speedup 1.0000x reference)
import functools

import jax
import jax.numpy as jnp
from jax import lax
from jax.experimental import pallas as pl
from jax.experimental.pallas import tpu as pltpu
from jax.experimental.pallas import tpu_sc as plsc

NUM_FACTORS = 32
BATCH = 16384
NC, NS = 2, 16
NW = NC * NS
BPW = BATCH // NW
L = 16
NSTREAM = BPW // 128


def _gmf_sc(users2d, items2d, u_tab, i_tab, w_flat, b_vec):
    mesh = plsc.VectorSubcoreMesh(core_axis_name="c", subcore_axis_name="s")

    @functools.partial(
        pl.kernel,
        mesh=mesh,
        compiler_params=pltpu.CompilerParams(
            needs_layout_passes=False, use_tc_tiling_on_sc=False),
        out_type=jax.ShapeDtypeStruct((BATCH,), jnp.float32),
        scratch_types=[
            pltpu.VMEM((NSTREAM, 128), jnp.int32),
            pltpu.VMEM((NSTREAM, 128), jnp.int32),
            pltpu.VMEM((NUM_FACTORS,), jnp.float32),
            pltpu.VMEM((L,), jnp.float32),
            pltpu.VMEM((NSTREAM, 128, NUM_FACTORS), jnp.float32),
            pltpu.VMEM((NSTREAM, 128, NUM_FACTORS), jnp.float32),
            pltpu.VMEM((BPW,), jnp.float32),
            pltpu.SemaphoreType.DMA,
        ],
    )
    def k(u_hbm, i_hbm, ut_hbm, it_hbm, w_hbm, b_hbm, o_hbm,
          uidx_v, iidx_v, wv, bv, urows, irows, acc, sem):
        wid = lax.axis_index("s") * NC + lax.axis_index("c")
        pltpu.sync_copy(u_hbm.at[pl.ds(wid * NSTREAM, NSTREAM)], uidx_v)
        pltpu.sync_copy(i_hbm.at[pl.ds(wid * NSTREAM, NSTREAM)], iidx_v)
        pltpu.sync_copy(w_hbm, wv)
        pltpu.sync_copy(b_hbm, bv)

        for j in range(NSTREAM):
            pltpu.async_copy(ut_hbm.at[uidx_v.at[j]], urows.at[j], sem)
            pltpu.async_copy(it_hbm.at[iidx_v.at[j]], irows.at[j], sem)
        for j in range(NSTREAM):
            pltpu.make_async_copy(
                ut_hbm.at[uidx_v.at[j]], urows.at[j], sem).wait()
            pltpu.make_async_copy(
                it_hbm.at[iidx_v.at[j]], irows.at[j], sem).wait()

        bias = bv[pl.ds(0, L)]
        w0 = wv[pl.ds(0, L)]
        w1 = wv[pl.ds(L, L)]
        iot = lax.iota(jnp.int32, L)

        @pl.loop(0, BPW, step=L)
        def _(c0):
            e = c0 + iot
            d0 = lax.shift_right_logical(e, 7)
            d1 = e & 127
            accv = jnp.zeros((L,), jnp.float32)
            for f in range(NUM_FACTORS):
                col = jnp.full((L,), f, jnp.int32)
                uf = plsc.load_gather(urows, [d0, d1, col])
                vf = plsc.load_gather(irows, [d0, d1, col])
                wf = w0[f] if f < L else w1[f - L]
                accv = accv + uf * vf * wf
            acc[pl.ds(c0, L)] = 1.0 / (1.0 + jnp.exp(-(accv + bias)))

        pltpu.sync_copy(acc, o_hbm.at[pl.ds(wid * BPW, BPW)])

    return k(users2d, items2d, u_tab, i_tab, w_flat, b_vec)


def kernel(users, items, user_table, item_table, W, b):
    users2d = users.astype(jnp.int32).reshape(BATCH // 128, 128)
    items2d = items.astype(jnp.int32).reshape(BATCH // 128, 128)
    b_vec = jnp.broadcast_to(b.astype(jnp.float32), (L,))
    return _gmf_sc(users2d, items2d, user_table, item_table,
                   W.reshape(-1), b_vec)


# speedup vs baseline: 3.7020x; 3.7020x over previous
import functools

import jax
import jax.numpy as jnp
from jax import lax
from jax.experimental import pallas as pl
from jax.experimental.pallas import tpu as pltpu
from jax.experimental.pallas import tpu_sc as plsc

NUM_FACTORS = 32
BATCH = 16384
NC, NS = 2, 16
NW = NC * NS
BPW = BATCH // NW
L = 16
TW = 128


def _gmf_sc(users, items, ut_t, it_t, w_flat, b_vec):
    mesh = plsc.VectorSubcoreMesh(core_axis_name="c", subcore_axis_name="s")

    @functools.partial(
        pl.kernel,
        mesh=mesh,
        compiler_params=pltpu.CompilerParams(needs_layout_passes=False),
        out_type=jax.ShapeDtypeStruct((BATCH,), jnp.float32),
        scratch_types=[
            pltpu.VMEM((BPW,), jnp.int32),
            pltpu.VMEM((BPW,), jnp.int32),
            pltpu.VMEM((NUM_FACTORS,), jnp.float32),
            pltpu.VMEM((L,), jnp.float32),
            pltpu.VMEM((L, NUM_FACTORS, TW), jnp.float32),
            pltpu.VMEM((NUM_FACTORS, L), jnp.float32),
            pltpu.VMEM((NUM_FACTORS, L), jnp.float32),
            pltpu.VMEM((BPW,), jnp.float32),
            pltpu.SemaphoreType.DMA,
        ],
    )
    def k(u_hbm, i_hbm, ut_hbm, it_hbm, w_hbm, b_hbm, o_hbm,
          uidx_v, iidx_v, wv, bv, blk, uext, iext, acc, sem):
        wid = lax.axis_index("s") * NC + lax.axis_index("c")
        base = wid * BPW
        pltpu.sync_copy(u_hbm.at[pl.ds(base, BPW)], uidx_v)
        pltpu.sync_copy(i_hbm.at[pl.ds(base, BPW)], iidx_v)
        pltpu.sync_copy(w_hbm, wv)
        pltpu.sync_copy(b_hbm, bv)

        bias = bv[pl.ds(0, L)]
        w0 = wv[pl.ds(0, L)]
        w1 = wv[pl.ds(L, L)]
        rows = lax.iota(jnp.int32, L)

        def fetch_and_extract(tab_hbm, idx_v, ext, c0):
            idx_vec = idx_v[pl.ds(c0, L)]
            for jc in range(L):
                uo = pl.multiple_of(idx_vec[jc] & ~(TW - 1), TW)
                pltpu.async_copy(tab_hbm.at[:, pl.ds(uo, TW)], blk.at[jc], sem)

            @pl.loop(0, L)
            def _(jc):
                pltpu.make_async_copy(
                    tab_hbm.at[:, pl.ds(0, TW)], blk.at[jc], sem).wait()

            lanes = idx_vec & (TW - 1)
            for f in range(NUM_FACTORS):
                col = jnp.full((L,), f, jnp.int32)
                ext[f, :] = plsc.load_gather(blk, [rows, col, lanes])

        @pl.loop(0, BPW, step=L)
        def _(c0):
            fetch_and_extract(ut_hbm, uidx_v, uext, c0)
            fetch_and_extract(it_hbm, iidx_v, iext, c0)

            accv = jnp.zeros((L,), jnp.float32)
            for f in range(NUM_FACTORS):
                wf = w0[f] if f < L else w1[f - L]
                accv = accv + uext[f, :] * iext[f, :] * wf
            acc[pl.ds(c0, L)] = 1.0 / (1.0 + jnp.exp(-(accv + bias)))

        pltpu.sync_copy(acc, o_hbm.at[pl.ds(base, BPW)])

    return k(users, items, ut_t, it_t, w_flat, b_vec)


def kernel(users, items, user_table, item_table, W, b):
    users = users.astype(jnp.int32)
    items = items.astype(jnp.int32)
    b_vec = jnp.broadcast_to(b.astype(jnp.float32), (L,))
    return _gmf_sc(users, items, user_table.T, item_table.T,
                   W.reshape(-1), b_vec)
